# baseline (speedup 1.0000x reference)
import functools

import jax
import jax.numpy as jnp
from jax import lax
from jax.experimental import pallas as pl
from jax.experimental.pallas import tpu as pltpu
from jax.experimental.pallas import tpu_sc as plsc

N = 10000
D = 128
H1 = 8
C1 = 8
HC = H1 * C1
NCLS = 40
NR = 10240
RB = 512
GRID = NR // RB
WS1 = 80
WD = 16
WS2 = 48
NW = 32
B = 128
SPW = NR // 16


def _tc_layer1(x_ref, w_ref, a_ref, ts_ref, td_ref):
    h = jnp.dot(x_ref[...], w_ref[...], preferred_element_type=jnp.float32)
    aa = jnp.dot(h, a_ref[...], preferred_element_type=jnp.float32)
    z8 = jnp.zeros((RB, 8), jnp.float32)
    ts_ref[...] = jnp.concatenate([h, aa[:, :H1], z8], axis=1)
    td_ref[...] = jnp.concatenate([aa[:, H1:], z8], axis=1)


def _tc_layer2(acc_ref, b1_ref, e8_ref, w2_ref, a2_ref, ts_ref, td_ref):
    a = acc_ref[0] + acc_ref[1]
    num = a[:, :HC]
    den = a[:, HC:HC + H1]
    r = 1.0 / (den + 1e-16)
    o = num * jnp.dot(r, e8_ref[...], preferred_element_type=jnp.float32)
    o = o + b1_ref[...]
    o = jnp.where(o >= 0.0, o, 0.2 * o)
    h2 = jnp.dot(o, w2_ref[...], preferred_element_type=jnp.float32)
    aa = jnp.dot(h2, a2_ref[...], preferred_element_type=jnp.float32)
    ts_ref[...] = jnp.concatenate(
        [aa[:, 0:1], jnp.zeros((RB, 7), jnp.float32), h2], axis=1)
    td_ref[...] = jnp.concatenate(
        [aa[:, 1:2], jnp.zeros((RB, 15), jnp.float32)], axis=1)


def _tc_final(acc_ref, b2_ref, out_ref):
    a = acc_ref[0] + acc_ref[1]
    den = a[:, 0:1]
    num = a[:, 8:8 + NCLS]
    o = num / (den + 1e-16) + b2_ref[...]
    m = jnp.max(o, axis=1, keepdims=True)
    s = o - m
    out_ref[...] = s - jnp.log(jnp.sum(jnp.exp(s), axis=1, keepdims=True))


def _make_edge_kernel(ws, ch, per_edge):
    epw = ch * B
    mesh = plsc.VectorSubcoreMesh(core_axis_name="c", subcore_axis_name="s")

    npairs = ch // 2

    @functools.partial(
        pl.kernel,
        out_type=jax.ShapeDtypeStruct((2, NR, ws), jnp.float32),
        mesh=mesh,
        scratch_types=[
            pltpu.VMEM_SHARED((NR, ws), jnp.float32),
            (pltpu.VMEM((B,), jnp.int32),) * 2,
            (pltpu.VMEM((B,), jnp.int32),) * 2,
            (pltpu.VMEM((B, ws), jnp.float32),) * 2,
            (pltpu.VMEM((B, WD), jnp.float32),) * 2,
            (pltpu.VMEM((B, ws), jnp.float32),) * 2,
            pltpu.VMEM((4, 16), jnp.float32),
            (pltpu.SemaphoreType.DMA,) * 2,
            (pltpu.SemaphoreType.DMA,) * 2,
        ],
        compiler_params=pltpu.CompilerParams(
            use_tc_tiling_on_sc=False, needs_layout_passes=False),
    )
    def k(tsrc, tad, srcl, dstl, zrows, acc_out,
          acc_sh, sidx, didx, srows, drows, msg, exb, semg, sems):
        c = lax.axis_index("c")
        s = lax.axis_index("s")
        wid = s * 2 + c
        pltpu.sync_copy(zrows.at[pl.ds(s * SPW, SPW)],
                        acc_sh.at[pl.ds(s * SPW, SPW)])
        plsc.subcore_barrier()

        def chunk(t, carry):
            base = pl.multiple_of((wid * ch + t) * B, B)
            pltpu.sync_copy(srcl.at[pl.ds(base, B)], sidx[0])
            pltpu.sync_copy(dstl.at[pl.ds(base, B)], didx[0])
            g0 = pltpu.async_copy(tsrc.at[sidx[0]], srows[0], semg[0])
            g1 = pltpu.async_copy(tad.at[didx[0]], drows[0], semg[0])
            g0.wait()
            g1.wait()

            def edge(e, carry2):
                per_edge(2 * e, srows[0], drows[0], msg[0], exb.at[0])
                per_edge(2 * e + 1, srows[0], drows[0], msg[0], exb.at[0])
                return carry2

            lax.fori_loop(0, B // 2, edge, 0)
            pltpu.sync_copy(msg[0], acc_sh.at[didx[0]], add=True)
            return carry

        lax.fori_loop(0, ch, chunk, 0)
        plsc.subcore_barrier()
        pltpu.sync_copy(acc_sh.at[pl.ds(s * SPW, SPW)],
                        acc_out.at[c, pl.ds(s * SPW, SPW)])

    return k


_GDN = lax.GatherDimensionNumbers(
    offset_dims=(), collapsed_slice_dims=(0,), start_index_map=(0,))


def _vgather(x, idx):
    return lax.gather(x, idx[:, None], _GDN, (1,),
                      mode=lax.GatherScatterMode.PROMISE_IN_BOUNDS)


def _edge1(e, srows, drows, msg, exb):
    ad = drows[e, :]
    av = srows[e, pl.ds(HC, 16)]
    al = av + ad
    al = jnp.where(al >= 0.0, al, 0.2 * al)
    ex = jnp.exp(al)
    hbase = lax.broadcasted_iota(jnp.int32, (16,), 0) // 8
    for j in range(4):
        sc = _vgather(ex, hbase + 2 * j)
        msg[e, pl.ds(j * 16, 16)] = srows[e, pl.ds(j * 16, 16)] * sc
    msg[e, pl.ds(HC, 16)] = ex


def _edge2(e, srows, drows, msg, exb):
    ad = drows[e, :]
    s0 = srows[e, pl.ds(0, 16)]
    al = s0 + ad
    al = jnp.where(al >= 0.0, al, 0.2 * al)
    ex = jnp.exp(al)
    bex = _vgather(ex, jnp.zeros((16,), jnp.int32))
    lane = lax.broadcasted_iota(jnp.int32, (16,), 0)
    one = jnp.full((16,), 1.0, jnp.float32)
    msg[e, pl.ds(0, 16)] = bex * jnp.where(lane == 0, one, s0)
    msg[e, pl.ds(16, 16)] = bex * srows[e, pl.ds(16, 16)]
    msg[e, pl.ds(32, 16)] = bex * srows[e, pl.ds(32, 16)]


def kernel(x, edge_index, W1, att_src1, att_dst1, b1,
           W2, att_src2, att_dst2, b2):
    E = edge_index.shape[1]
    el = E + N
    ch = -(-el // (NW * B))
    epad = ch * B * NW

    xp = jnp.pad(x, ((0, NR - N), (0, 0)))
    eye8 = jnp.eye(H1, dtype=jnp.float32)
    a_s = (eye8[:, None, :] * att_src1[:, :, None]).reshape(HC, H1)
    a_d = (eye8[:, None, :] * att_dst1[:, :, None]).reshape(HC, H1)
    a1 = jnp.concatenate([a_s, a_d], axis=1)
    e8 = jnp.kron(eye8, jnp.ones((1, C1), jnp.float32))
    a2 = jnp.concatenate([att_src2.T, att_dst2.T], axis=1)
    pad_e = epad - el
    loop = jnp.arange(N, dtype=jnp.int32)
    dummy = jnp.full((pad_e,), N, dtype=jnp.int32)
    srcl = jnp.concatenate([edge_index[0], loop, dummy])
    dstl = jnp.concatenate([edge_index[1], loop, dummy])
    z1 = jnp.zeros((NR, WS1), jnp.float32)
    z2 = jnp.zeros((NR, WS2), jnp.float32)

    ts1, td1 = pl.pallas_call(
        _tc_layer1,
        grid=(GRID,),
        in_specs=[
            pl.BlockSpec((RB, D), lambda i: (i, 0)),
            pl.BlockSpec((D, HC), lambda i: (0, 0)),
            pl.BlockSpec((HC, 16), lambda i: (0, 0)),
        ],
        out_specs=[
            pl.BlockSpec((RB, WS1), lambda i: (i, 0)),
            pl.BlockSpec((RB, WD), lambda i: (i, 0)),
        ],
        out_shape=[
            jax.ShapeDtypeStruct((NR, WS1), jnp.float32),
            jax.ShapeDtypeStruct((NR, WD), jnp.float32),
        ],
    )(xp, W1, a1)

    acc1 = _make_edge_kernel(WS1, ch, _edge1)(ts1, td1, srcl, dstl, z1)

    ts2, td2 = pl.pallas_call(
        _tc_layer2,
        grid=(GRID,),
        in_specs=[
            pl.BlockSpec((2, RB, WS1), lambda i: (0, i, 0)),
            pl.BlockSpec((1, HC), lambda i: (0, 0)),
            pl.BlockSpec((H1, HC), lambda i: (0, 0)),
            pl.BlockSpec((HC, NCLS), lambda i: (0, 0)),
            pl.BlockSpec((NCLS, 2), lambda i: (0, 0)),
        ],
        out_specs=[
            pl.BlockSpec((RB, WS2), lambda i: (i, 0)),
            pl.BlockSpec((RB, WD), lambda i: (i, 0)),
        ],
        out_shape=[
            jax.ShapeDtypeStruct((NR, WS2), jnp.float32),
            jax.ShapeDtypeStruct((NR, WD), jnp.float32),
        ],
    )(acc1, b1.reshape(1, HC), e8, W2, a2)

    acc2 = _make_edge_kernel(WS2, ch, _edge2)(ts2, td2, srcl, dstl, z2)

    out = pl.pallas_call(
        _tc_final,
        grid=(GRID,),
        in_specs=[
            pl.BlockSpec((2, RB, WS2), lambda i: (0, i, 0)),
            pl.BlockSpec((1, NCLS), lambda i: (0, 0)),
        ],
        out_specs=pl.BlockSpec((RB, NCLS), lambda i: (i, 0)),
        out_shape=jax.ShapeDtypeStruct((NR, NCLS), jnp.float32),
    )(acc2, b2.reshape(1, NCLS))

    return out[:N]


# speedup vs baseline: 1.3452x; 1.3452x over previous
import functools

import jax
import jax.numpy as jnp
from jax import lax
from jax.experimental import pallas as pl
from jax.experimental.pallas import tpu as pltpu
from jax.experimental.pallas import tpu_sc as plsc

N = 10000
D = 128
H1 = 8
C1 = 8
HC = H1 * C1
NCLS = 40
NR = 10240
RB = 512
GRID = NR // RB
WS1 = 80
WD = 16
WS2 = 48
NW = 32
B = 128
SPW = NR // 16


def _tc_layer1(x_ref, w_ref, a_ref, ts_ref, td_ref):
    h = jnp.dot(x_ref[...], w_ref[...], preferred_element_type=jnp.float32)
    aa = jnp.dot(h, a_ref[...], preferred_element_type=jnp.float32)
    z8 = jnp.zeros((RB, 8), jnp.float32)
    ts_ref[...] = jnp.concatenate([h, aa[:, :H1], z8], axis=1)
    td_ref[...] = jnp.concatenate([aa[:, H1:], z8], axis=1)


def _tc_layer2(acc_ref, b1_ref, e8_ref, w2_ref, a2_ref, ts_ref, td_ref):
    a = acc_ref[0] + acc_ref[1]
    num = a[:, :HC]
    den = a[:, HC:HC + H1]
    r = 1.0 / (den + 1e-16)
    o = num * jnp.dot(r, e8_ref[...], preferred_element_type=jnp.float32)
    o = o + b1_ref[...]
    o = jnp.where(o >= 0.0, o, 0.2 * o)
    h2 = jnp.dot(o, w2_ref[...], preferred_element_type=jnp.float32)
    aa = jnp.dot(h2, a2_ref[...], preferred_element_type=jnp.float32)
    ts_ref[...] = jnp.concatenate(
        [aa[:, 0:1], jnp.zeros((RB, 7), jnp.float32), h2], axis=1)
    td_ref[...] = jnp.concatenate(
        [aa[:, 1:2], jnp.zeros((RB, 15), jnp.float32)], axis=1)


def _tc_final(acc_ref, b2_ref, out_ref):
    a = acc_ref[0] + acc_ref[1]
    den = a[:, 0:1]
    num = a[:, 8:8 + NCLS]
    o = num / (den + 1e-16) + b2_ref[...]
    m = jnp.max(o, axis=1, keepdims=True)
    s = o - m
    out_ref[...] = s - jnp.log(jnp.sum(jnp.exp(s), axis=1, keepdims=True))


def _make_edge_kernel(ws, ch, per_edge):
    epw = ch * B
    mesh = plsc.VectorSubcoreMesh(core_axis_name="c", subcore_axis_name="s")

    @functools.partial(
        pl.kernel,
        out_type=jax.ShapeDtypeStruct((2, NR, ws), jnp.float32),
        mesh=mesh,
        scratch_types=[
            pltpu.VMEM_SHARED((NR, ws), jnp.float32),
            pltpu.VMEM((ch, B), jnp.int32),
            pltpu.VMEM((ch, B), jnp.int32),
            (pltpu.VMEM((B, ws), jnp.float32),) * 2,
            (pltpu.VMEM((B, WD), jnp.float32),) * 2,
            (pltpu.VMEM((B, ws), jnp.float32),) * 2,
            (pltpu.SemaphoreType.DMA,) * 2,
            (pltpu.SemaphoreType.DMA,) * 2,
        ],
        compiler_params=pltpu.CompilerParams(
            use_tc_tiling_on_sc=False, needs_layout_passes=False),
    )
    def k(tsrc, tad, srcl, dstl, zrows, acc_out,
          acc_sh, sidxa, didxa, srows, drows, msg, semg, sems):
        c = lax.axis_index("c")
        s = lax.axis_index("s")
        wid = s * 2 + c
        pltpu.sync_copy(zrows.at[pl.ds(s * SPW, SPW)],
                        acc_sh.at[pl.ds(s * SPW, SPW)])
        pltpu.sync_copy(srcl.at[pl.ds(wid * ch, ch)], sidxa)
        pltpu.sync_copy(dstl.at[pl.ds(wid * ch, ch)], didxa)
        plsc.subcore_barrier()

        def issue_gathers(t, p):
            pltpu.async_copy(tsrc.at[sidxa.at[t]], srows[p], semg[p])
            pltpu.async_copy(tad.at[didxa.at[t]], drows[p], semg[p])

        def wait_gathers(t, p):
            pltpu.make_async_copy(
                tsrc.at[sidxa.at[t]], srows[p], semg[p]).wait()
            pltpu.make_async_copy(
                tad.at[didxa.at[t]], drows[p], semg[p]).wait()

        def compute(p):
            def edge(e, carry):
                per_edge(e, srows[p], drows[p], msg[p])
                return carry

            lax.fori_loop(0, B, edge, 0)

        def issue_scatter(t, p):
            pltpu.async_copy(msg[p], acc_sh.at[didxa.at[t]], sems[p],
                             add=True)

        def wait_scatter(t, p):
            pltpu.make_async_copy(
                msg[p], acc_sh.at[didxa.at[t]], sems[p]).wait()

        issue_gathers(0, 0)
        issue_gathers(1, 1)
        wait_gathers(0, 0)
        compute(0)
        issue_scatter(0, 0)

        def pair(i, carry):
            t = 2 * i + 1
            issue_gathers(t + 1, 0)
            wait_gathers(t, 1)
            compute(1)
            wait_scatter(t - 1, 0)
            issue_scatter(t, 1)
            issue_gathers(t + 2, 1)
            wait_gathers(t + 1, 0)
            compute(0)
            wait_scatter(t, 1)
            issue_scatter(t + 1, 0)
            return carry

        lax.fori_loop(0, (ch - 2) // 2, pair, 0)
        wait_gathers(ch - 1, 1)
        compute(1)
        wait_scatter(ch - 2, 0)
        issue_scatter(ch - 1, 1)
        wait_scatter(ch - 1, 1)
        plsc.subcore_barrier()
        pltpu.sync_copy(acc_sh.at[pl.ds(s * SPW, SPW)],
                        acc_out.at[c, pl.ds(s * SPW, SPW)])

    return k


_GDN = lax.GatherDimensionNumbers(
    offset_dims=(), collapsed_slice_dims=(0,), start_index_map=(0,))


def _vgather(x, idx):
    return lax.gather(x, idx[:, None], _GDN, (1,),
                      mode=lax.GatherScatterMode.PROMISE_IN_BOUNDS)


def _edge1(e, srows, drows, msg):
    ad = drows[e, :]
    av = srows[e, pl.ds(HC, 16)]
    al = av + ad
    al = jnp.where(al >= 0.0, al, 0.2 * al)
    ex = jnp.exp(al)
    hbase = lax.broadcasted_iota(jnp.int32, (16,), 0) // 8
    for j in range(4):
        sc = _vgather(ex, hbase + 2 * j)
        msg[e, pl.ds(j * 16, 16)] = srows[e, pl.ds(j * 16, 16)] * sc
    msg[e, pl.ds(HC, 16)] = ex


def _edge2(e, srows, drows, msg):
    ad = drows[e, :]
    s0 = srows[e, pl.ds(0, 16)]
    al = s0 + ad
    al = jnp.where(al >= 0.0, al, 0.2 * al)
    ex = jnp.exp(al)
    bex = _vgather(ex, jnp.zeros((16,), jnp.int32))
    lane = lax.broadcasted_iota(jnp.int32, (16,), 0)
    one = jnp.full((16,), 1.0, jnp.float32)
    msg[e, pl.ds(0, 16)] = bex * jnp.where(lane == 0, one, s0)
    msg[e, pl.ds(16, 16)] = bex * srows[e, pl.ds(16, 16)]
    msg[e, pl.ds(32, 16)] = bex * srows[e, pl.ds(32, 16)]


def kernel(x, edge_index, W1, att_src1, att_dst1, b1,
           W2, att_src2, att_dst2, b2):
    E = edge_index.shape[1]
    el = E + N
    ch = -(-el // (NW * B))
    ch += ch % 2
    epad = ch * B * NW

    xp = jnp.pad(x, ((0, NR - N), (0, 0)))
    eye8 = jnp.eye(H1, dtype=jnp.float32)
    a_s = (eye8[:, None, :] * att_src1[:, :, None]).reshape(HC, H1)
    a_d = (eye8[:, None, :] * att_dst1[:, :, None]).reshape(HC, H1)
    a1 = jnp.concatenate([a_s, a_d], axis=1)
    e8 = jnp.kron(eye8, jnp.ones((1, C1), jnp.float32))
    a2 = jnp.concatenate([att_src2.T, att_dst2.T], axis=1)
    pad_e = epad - el
    loop = jnp.arange(N, dtype=jnp.int32)
    dummy = jnp.full((pad_e,), N, dtype=jnp.int32)
    srcl = jnp.concatenate([edge_index[0], loop, dummy]).reshape(NW * ch, B)
    dstl = jnp.concatenate([edge_index[1], loop, dummy]).reshape(NW * ch, B)
    z1 = jnp.zeros((NR, WS1), jnp.float32)
    z2 = jnp.zeros((NR, WS2), jnp.float32)

    ts1, td1 = pl.pallas_call(
        _tc_layer1,
        grid=(GRID,),
        in_specs=[
            pl.BlockSpec((RB, D), lambda i: (i, 0)),
            pl.BlockSpec((D, HC), lambda i: (0, 0)),
            pl.BlockSpec((HC, 16), lambda i: (0, 0)),
        ],
        out_specs=[
            pl.BlockSpec((RB, WS1), lambda i: (i, 0)),
            pl.BlockSpec((RB, WD), lambda i: (i, 0)),
        ],
        out_shape=[
            jax.ShapeDtypeStruct((NR, WS1), jnp.float32),
            jax.ShapeDtypeStruct((NR, WD), jnp.float32),
        ],
    )(xp, W1, a1)

    acc1 = _make_edge_kernel(WS1, ch, _edge1)(ts1, td1, srcl, dstl, z1)

    ts2, td2 = pl.pallas_call(
        _tc_layer2,
        grid=(GRID,),
        in_specs=[
            pl.BlockSpec((2, RB, WS1), lambda i: (0, i, 0)),
            pl.BlockSpec((1, HC), lambda i: (0, 0)),
            pl.BlockSpec((H1, HC), lambda i: (0, 0)),
            pl.BlockSpec((HC, NCLS), lambda i: (0, 0)),
            pl.BlockSpec((NCLS, 2), lambda i: (0, 0)),
        ],
        out_specs=[
            pl.BlockSpec((RB, WS2), lambda i: (i, 0)),
            pl.BlockSpec((RB, WD), lambda i: (i, 0)),
        ],
        out_shape=[
            jax.ShapeDtypeStruct((NR, WS2), jnp.float32),
            jax.ShapeDtypeStruct((NR, WD), jnp.float32),
        ],
    )(acc1, b1.reshape(1, HC), e8, W2, a2)

    acc2 = _make_edge_kernel(WS2, ch, _edge2)(ts2, td2, srcl, dstl, z2)

    out = pl.pallas_call(
        _tc_final,
        grid=(GRID,),
        in_specs=[
            pl.BlockSpec((2, RB, WS2), lambda i: (0, i, 0)),
            pl.BlockSpec((1, NCLS), lambda i: (0, 0)),
        ],
        out_specs=pl.BlockSpec((RB, NCLS), lambda i: (i, 0)),
        out_shape=jax.ShapeDtypeStruct((NR, NCLS), jnp.float32),
    )(acc2, b2.reshape(1, NCLS))

    return out[:N]


# speedup vs baseline: 1.8734x; 1.3926x over previous
import functools

import jax
import jax.numpy as jnp
from jax import lax
from jax.experimental import pallas as pl
from jax.experimental.pallas import tpu as pltpu
from jax.experimental.pallas import tpu_sc as plsc

N = 10000
D = 128
H1 = 8
C1 = 8
HC = H1 * C1
NCLS = 40
NR = 10240
RB = 512
GRID = NR // RB
WS1 = 80
WD = 16
WS2 = 48
NW = 32
B = 128
SPW = NR // 16


def _tc_layer1(x_ref, w_ref, a_ref, ts_ref, td_ref):
    h = jnp.dot(x_ref[...], w_ref[...], preferred_element_type=jnp.float32)
    aa = jnp.dot(h, a_ref[...], preferred_element_type=jnp.float32)
    z8 = jnp.zeros((RB, 8), jnp.float32)
    ts_ref[...] = jnp.concatenate([h, aa[:, :H1], z8], axis=1)
    td_ref[...] = jnp.concatenate([aa[:, H1:], z8], axis=1)


def _tc_layer2(acc_ref, b1_ref, e8_ref, w2_ref, a2_ref, ts_ref, td_ref):
    a = acc_ref[0] + acc_ref[1]
    num = a[:, :HC]
    den = a[:, HC:HC + H1]
    r = 1.0 / (den + 1e-16)
    o = num * jnp.dot(r, e8_ref[...], preferred_element_type=jnp.float32)
    o = o + b1_ref[...]
    o = jnp.where(o >= 0.0, o, 0.2 * o)
    h2 = jnp.dot(o, w2_ref[...], preferred_element_type=jnp.float32)
    aa = jnp.dot(h2, a2_ref[...], preferred_element_type=jnp.float32)
    ts_ref[...] = jnp.concatenate(
        [aa[:, 0:1], jnp.zeros((RB, 7), jnp.float32), h2], axis=1)
    td_ref[...] = jnp.concatenate(
        [aa[:, 1:2], jnp.zeros((RB, 15), jnp.float32)], axis=1)


def _tc_final(acc_ref, b2_ref, out_ref):
    a = acc_ref[0] + acc_ref[1]
    den = a[:, 0:1]
    num = a[:, 8:8 + NCLS]
    o = num / (den + 1e-16) + b2_ref[...]
    m = jnp.max(o, axis=1, keepdims=True)
    s = o - m
    out_ref[...] = s - jnp.log(jnp.sum(jnp.exp(s), axis=1, keepdims=True))


def _make_edge_kernel(ws, ch, per_edge):
    epw = ch * B
    mesh = plsc.VectorSubcoreMesh(core_axis_name="c", subcore_axis_name="s")

    @functools.partial(
        pl.kernel,
        out_type=jax.ShapeDtypeStruct((2, NR, ws), jnp.float32),
        mesh=mesh,
        scratch_types=[
            pltpu.VMEM_SHARED((NR, ws), jnp.float32),
            pltpu.VMEM((ch, B), jnp.int32),
            pltpu.VMEM((ch, B), jnp.int32),
            (pltpu.VMEM((B, ws), jnp.float32),) * 2,
            (pltpu.VMEM((B, WD), jnp.float32),) * 2,
            (pltpu.VMEM((B, ws), jnp.float32),) * 2,
            (pltpu.SemaphoreType.DMA,) * 2,
            (pltpu.SemaphoreType.DMA,) * 2,
        ],
        compiler_params=pltpu.CompilerParams(
            use_tc_tiling_on_sc=False, needs_layout_passes=False),
    )
    def k(tsrc, tad, srcl, dstl, zrows, acc_out,
          acc_sh, sidxa, didxa, srows, drows, msg, semg, sems):
        c = lax.axis_index("c")
        s = lax.axis_index("s")
        wid = s * 2 + c
        pltpu.sync_copy(zrows.at[pl.ds(s * SPW, SPW)],
                        acc_sh.at[pl.ds(s * SPW, SPW)])
        pltpu.sync_copy(srcl.at[pl.ds(wid * ch, ch)], sidxa)
        pltpu.sync_copy(dstl.at[pl.ds(wid * ch, ch)], didxa)
        plsc.subcore_barrier()

        def issue_gathers(t, p):
            pltpu.async_copy(tsrc.at[sidxa.at[t]], srows[p], semg[p])
            pltpu.async_copy(tad.at[didxa.at[t]], drows[p], semg[p])

        def wait_gathers(t, p):
            pltpu.make_async_copy(
                tsrc.at[sidxa.at[t]], srows[p], semg[p]).wait()
            pltpu.make_async_copy(
                tad.at[didxa.at[t]], drows[p], semg[p]).wait()

        def compute(p):
            def edge(e, carry):
                per_edge(e, srows[p], drows[p], msg[p])
                return carry

            lax.fori_loop(0, B, edge, 0)

        def issue_scatter(t, p):
            pltpu.async_copy(msg[p], acc_sh.at[didxa.at[t]], sems[p],
                             add=True)

        def wait_scatter(t, p):
            pltpu.make_async_copy(
                msg[p], acc_sh.at[didxa.at[t]], sems[p]).wait()

        issue_gathers(0, 0)
        issue_gathers(1, 1)
        wait_gathers(0, 0)
        compute(0)
        issue_scatter(0, 0)

        def pair(i, carry):
            t = 2 * i + 1
            issue_gathers(t + 1, 0)
            wait_gathers(t, 1)
            compute(1)
            wait_scatter(t - 1, 0)
            issue_scatter(t, 1)
            issue_gathers(t + 2, 1)
            wait_gathers(t + 1, 0)
            compute(0)
            wait_scatter(t, 1)
            issue_scatter(t + 1, 0)
            return carry

        lax.fori_loop(0, (ch - 2) // 2, pair, 0)
        wait_gathers(ch - 1, 1)
        compute(1)
        wait_scatter(ch - 2, 0)
        issue_scatter(ch - 1, 1)
        wait_scatter(ch - 1, 1)
        plsc.subcore_barrier()
        pltpu.sync_copy(acc_sh.at[pl.ds(s * SPW, SPW)],
                        acc_out.at[c, pl.ds(s * SPW, SPW)])

    return k


_GDN = lax.GatherDimensionNumbers(
    offset_dims=(), collapsed_slice_dims=(0,), start_index_map=(0,))


def _vgather(x, idx):
    return lax.gather(x, idx[:, None], _GDN, (1,),
                      mode=lax.GatherScatterMode.PROMISE_IN_BOUNDS)


def _edge1(e, srows, drows, msg):
    ad = drows[e, :]
    av = srows[e, pl.ds(HC, 16)]
    al = av + ad
    al = jnp.where(al >= 0.0, al, 0.2 * al)
    ex = jnp.exp(al)
    hbase = lax.broadcasted_iota(jnp.int32, (16,), 0) // 8
    for j in range(4):
        sc = _vgather(ex, hbase + 2 * j)
        msg[e, pl.ds(j * 16, 16)] = srows[e, pl.ds(j * 16, 16)] * sc
    msg[e, pl.ds(HC, 16)] = ex


def _edge2(e, srows, drows, msg):
    ad = drows[e, :]
    s0 = srows[e, pl.ds(0, 16)]
    al = s0 + ad
    al = jnp.where(al >= 0.0, al, 0.2 * al)
    ex = jnp.exp(al)
    bex = _vgather(ex, jnp.zeros((16,), jnp.int32))
    lane = lax.broadcasted_iota(jnp.int32, (16,), 0)
    one = jnp.full((16,), 1.0, jnp.float32)
    msg[e, pl.ds(0, 16)] = bex * jnp.where(lane == 0, one, s0)
    msg[e, pl.ds(16, 16)] = bex * srows[e, pl.ds(16, 16)]
    msg[e, pl.ds(32, 16)] = bex * srows[e, pl.ds(32, 16)]


def kernel(x, edge_index, W1, att_src1, att_dst1, b1,
           W2, att_src2, att_dst2, b2):
    E = edge_index.shape[1]
    el = E + N
    ch = -(-el // (NW * B))
    ch += ch % 2
    epad = ch * B * NW

    xp = jnp.pad(x, ((0, NR - N), (0, 0)))
    eye8 = jnp.eye(H1, dtype=jnp.float32)
    a_s = (eye8[:, None, :] * att_src1[:, :, None]).reshape(HC, H1)
    a_d = (eye8[:, None, :] * att_dst1[:, :, None]).reshape(HC, H1)
    a1 = jnp.concatenate([a_s, a_d], axis=1)
    e8 = jnp.kron(eye8, jnp.ones((1, C1), jnp.float32))
    a2 = jnp.concatenate([att_src2.T, att_dst2.T], axis=1)
    pad_e = epad - el
    loop = jnp.arange(N, dtype=jnp.int32)
    dummy = N + jnp.arange(pad_e, dtype=jnp.int32) % (NR - N)
    srcl = jnp.concatenate([edge_index[0], loop, dummy]).reshape(NW * ch, B)
    dstl = jnp.concatenate([edge_index[1], loop, dummy]).reshape(NW * ch, B)
    z1 = jnp.zeros((NR, WS1), jnp.float32)
    z2 = jnp.zeros((NR, WS2), jnp.float32)

    ts1, td1 = pl.pallas_call(
        _tc_layer1,
        grid=(GRID,),
        in_specs=[
            pl.BlockSpec((RB, D), lambda i: (i, 0)),
            pl.BlockSpec((D, HC), lambda i: (0, 0)),
            pl.BlockSpec((HC, 16), lambda i: (0, 0)),
        ],
        out_specs=[
            pl.BlockSpec((RB, WS1), lambda i: (i, 0)),
            pl.BlockSpec((RB, WD), lambda i: (i, 0)),
        ],
        out_shape=[
            jax.ShapeDtypeStruct((NR, WS1), jnp.float32),
            jax.ShapeDtypeStruct((NR, WD), jnp.float32),
        ],
    )(xp, W1, a1)

    acc1 = _make_edge_kernel(WS1, ch, _edge1)(ts1, td1, srcl, dstl, z1)

    ts2, td2 = pl.pallas_call(
        _tc_layer2,
        grid=(GRID,),
        in_specs=[
            pl.BlockSpec((2, RB, WS1), lambda i: (0, i, 0)),
            pl.BlockSpec((1, HC), lambda i: (0, 0)),
            pl.BlockSpec((H1, HC), lambda i: (0, 0)),
            pl.BlockSpec((HC, NCLS), lambda i: (0, 0)),
            pl.BlockSpec((NCLS, 2), lambda i: (0, 0)),
        ],
        out_specs=[
            pl.BlockSpec((RB, WS2), lambda i: (i, 0)),
            pl.BlockSpec((RB, WD), lambda i: (i, 0)),
        ],
        out_shape=[
            jax.ShapeDtypeStruct((NR, WS2), jnp.float32),
            jax.ShapeDtypeStruct((NR, WD), jnp.float32),
        ],
    )(acc1, b1.reshape(1, HC), e8, W2, a2)

    acc2 = _make_edge_kernel(WS2, ch, _edge2)(ts2, td2, srcl, dstl, z2)

    out = pl.pallas_call(
        _tc_final,
        grid=(GRID,),
        in_specs=[
            pl.BlockSpec((2, RB, WS2), lambda i: (0, i, 0)),
            pl.BlockSpec((1, NCLS), lambda i: (0, 0)),
        ],
        out_specs=pl.BlockSpec((RB, NCLS), lambda i: (i, 0)),
        out_shape=jax.ShapeDtypeStruct((NR, NCLS), jnp.float32),
    )(acc2, b2.reshape(1, NCLS))

    return out[:N]


# speedup vs baseline: 1.9095x; 1.0193x over previous
import functools

import jax
import jax.numpy as jnp
from jax import lax
from jax.experimental import pallas as pl
from jax.experimental.pallas import tpu as pltpu
from jax.experimental.pallas import tpu_sc as plsc

N = 10000
D = 128
H1 = 8
C1 = 8
HC = H1 * C1
NCLS = 40
RB = 400
GRID = N // RB
WS1 = 80
WD = 16
WS2 = 48
NW = 32
B = 128
SPW = N // 16
ROWS = 2500
CHB = ROWS // NW
XTRA = ROWS - CHB * NW
CHMAX = CHB + XTRA // 2


def _lrelu(x):
    return jnp.where(x >= 0.0, x, 0.2 * x)


def _tc_layer1(x_ref, w_ref, a_ref, ts_ref, td_ref):
    h = jnp.dot(x_ref[...], w_ref[...], preferred_element_type=jnp.float32)
    aa = jnp.dot(h, a_ref[...], preferred_element_type=jnp.float32)
    z8 = jnp.zeros((RB, 8), jnp.float32)
    ts_ref[...] = jnp.concatenate([h, aa[:, :H1], z8], axis=1)
    td_ref[...] = jnp.concatenate([aa[:, H1:], z8], axis=1)


def _tc_layer2(acc_ref, ts1_ref, td1_ref, b1_ref, e8_ref, w2_ref, a2_ref,
               ts_ref, td_ref):
    a = acc_ref[0] + acc_ref[1]
    h = ts1_ref[:, :HC]
    exs = jnp.exp(_lrelu(ts1_ref[:, HC:HC + H1] + td1_ref[:, :H1]))
    den = a[:, HC:HC + H1] + exs
    num = a[:, :HC] + h * jnp.dot(exs, e8_ref[...],
                                  preferred_element_type=jnp.float32)
    r = 1.0 / (den + 1e-16)
    o = num * jnp.dot(r, e8_ref[...], preferred_element_type=jnp.float32)
    o = _lrelu(o + b1_ref[...])
    h2 = jnp.dot(o, w2_ref[...], preferred_element_type=jnp.float32)
    aa = jnp.dot(h2, a2_ref[...], preferred_element_type=jnp.float32)
    ts_ref[...] = jnp.concatenate(
        [aa[:, 0:1], jnp.zeros((RB, 7), jnp.float32), h2], axis=1)
    td_ref[...] = jnp.concatenate(
        [aa[:, 1:2], jnp.zeros((RB, 15), jnp.float32)], axis=1)


def _tc_final(acc_ref, ts2_ref, td2_ref, b2_ref, out_ref):
    a = acc_ref[0] + acc_ref[1]
    h2 = ts2_ref[:, 8:8 + NCLS]
    exs = jnp.exp(_lrelu(ts2_ref[:, 0:1] + td2_ref[:, 0:1]))
    den = a[:, 0:1] + exs
    num = a[:, 8:8 + NCLS] + h2 * exs
    o = num / (den + 1e-16) + b2_ref[...]
    m = jnp.max(o, axis=1, keepdims=True)
    s = o - m
    out_ref[...] = s - jnp.log(jnp.sum(jnp.exp(s), axis=1, keepdims=True))


def _make_edge_kernel(ws, per_edge):
    mesh = plsc.VectorSubcoreMesh(core_axis_name="c", subcore_axis_name="s")

    @functools.partial(
        pl.kernel,
        out_type=jax.ShapeDtypeStruct((2, N, ws), jnp.float32),
        mesh=mesh,
        scratch_types=[
            pltpu.VMEM_SHARED((N, ws), jnp.float32),
            pltpu.VMEM((CHMAX, B), jnp.int32),
            pltpu.VMEM((CHMAX, B), jnp.int32),
            (pltpu.VMEM((B, ws), jnp.float32),) * 2,
            (pltpu.VMEM((B, WD), jnp.float32),) * 2,
            (pltpu.VMEM((B, ws), jnp.float32),) * 2,
            (pltpu.SemaphoreType.DMA,) * 2,
            (pltpu.SemaphoreType.DMA,) * 2,
        ],
        compiler_params=pltpu.CompilerParams(
            use_tc_tiling_on_sc=False, needs_layout_passes=False),
    )
    def k(tsrc, tad, srcl, dstl, zrows, acc_out,
          acc_sh, sidxa, didxa, srows, drows, msg, semg, sems):
        c = lax.axis_index("c")
        s = lax.axis_index("s")
        wid = s * 2 + c
        start = CHB * wid + 2 * jnp.minimum(wid, XTRA // 2)
        nch = CHB + jnp.where(wid < XTRA // 2, XTRA // 2, 0)
        pltpu.sync_copy(zrows.at[pl.ds(s * SPW, SPW)],
                        acc_sh.at[pl.ds(s * SPW, SPW)])
        pltpu.sync_copy(srcl.at[pl.ds(start, CHMAX)], sidxa)
        pltpu.sync_copy(dstl.at[pl.ds(start, CHMAX)], didxa)
        plsc.subcore_barrier()

        def issue_gathers(t, p):
            pltpu.async_copy(tsrc.at[sidxa.at[t]], srows[p], semg[p])
            pltpu.async_copy(tad.at[didxa.at[t]], drows[p], semg[p])

        def wait_gathers(t, p):
            pltpu.make_async_copy(
                tsrc.at[sidxa.at[t]], srows[p], semg[p]).wait()
            pltpu.make_async_copy(
                tad.at[didxa.at[t]], drows[p], semg[p]).wait()

        def compute(p):
            def edge(e, carry):
                per_edge(e, srows[p], drows[p], msg[p])
                return carry

            lax.fori_loop(0, B, edge, 0)

        def issue_scatter(t, p):
            pltpu.async_copy(msg[p], acc_sh.at[didxa.at[t]], sems[p],
                             add=True)

        def wait_scatter(t, p):
            pltpu.make_async_copy(
                msg[p], acc_sh.at[didxa.at[t]], sems[p]).wait()

        issue_gathers(0, 0)
        issue_gathers(1, 1)
        wait_gathers(0, 0)
        compute(0)
        issue_scatter(0, 0)

        def pair(i, carry):
            t = 2 * i + 1
            issue_gathers(t + 1, 0)
            wait_gathers(t, 1)
            compute(1)
            wait_scatter(t - 1, 0)
            issue_scatter(t, 1)
            issue_gathers(t + 2, 1)
            wait_gathers(t + 1, 0)
            compute(0)
            wait_scatter(t, 1)
            issue_scatter(t + 1, 0)
            return carry

        lax.fori_loop(0, (nch - 2) // 2, pair, 0)
        wait_gathers(nch - 1, 1)
        compute(1)
        wait_scatter(nch - 2, 0)
        issue_scatter(nch - 1, 1)
        wait_scatter(nch - 1, 1)
        plsc.subcore_barrier()
        pltpu.sync_copy(acc_sh.at[pl.ds(s * SPW, SPW)],
                        acc_out.at[c, pl.ds(s * SPW, SPW)])

    return k


_GDN = lax.GatherDimensionNumbers(
    offset_dims=(), collapsed_slice_dims=(0,), start_index_map=(0,))


def _vgather(x, idx):
    return lax.gather(x, idx[:, None], _GDN, (1,),
                      mode=lax.GatherScatterMode.PROMISE_IN_BOUNDS)


def _edge1(e, srows, drows, msg):
    ad = drows[e, :]
    av = srows[e, pl.ds(HC, 16)]
    al = av + ad
    al = jnp.where(al >= 0.0, al, 0.2 * al)
    ex = jnp.exp(al)
    hbase = lax.broadcasted_iota(jnp.int32, (16,), 0) // 8
    for j in range(4):
        sc = _vgather(ex, hbase + 2 * j)
        msg[e, pl.ds(j * 16, 16)] = srows[e, pl.ds(j * 16, 16)] * sc
    msg[e, pl.ds(HC, 16)] = ex


def _edge2(e, srows, drows, msg):
    ad = drows[e, :]
    s0 = srows[e, pl.ds(0, 16)]
    al = s0 + ad
    al = jnp.where(al >= 0.0, al, 0.2 * al)
    ex = jnp.exp(al)
    bex = _vgather(ex, jnp.zeros((16,), jnp.int32))
    lane = lax.broadcasted_iota(jnp.int32, (16,), 0)
    one = jnp.full((16,), 1.0, jnp.float32)
    msg[e, pl.ds(0, 16)] = bex * jnp.where(lane == 0, one, s0)
    msg[e, pl.ds(16, 16)] = bex * srows[e, pl.ds(16, 16)]
    msg[e, pl.ds(32, 16)] = bex * srows[e, pl.ds(32, 16)]


def kernel(x, edge_index, W1, att_src1, att_dst1, b1,
           W2, att_src2, att_dst2, b2):
    eye8 = jnp.eye(H1, dtype=jnp.float32)
    a_s = (eye8[:, None, :] * att_src1[:, :, None]).reshape(HC, H1)
    a_d = (eye8[:, None, :] * att_dst1[:, :, None]).reshape(HC, H1)
    a1 = jnp.concatenate([a_s, a_d], axis=1)
    e8 = jnp.kron(eye8, jnp.ones((1, C1), jnp.float32))
    a2 = jnp.concatenate([att_src2.T, att_dst2.T], axis=1)
    srcl = jnp.pad(edge_index[0].reshape(ROWS, B), ((0, 2), (0, 0)))
    dstl = jnp.pad(edge_index[1].reshape(ROWS, B), ((0, 2), (0, 0)))
    z1 = jnp.zeros((N, WS1), jnp.float32)
    z2 = jnp.zeros((N, WS2), jnp.float32)

    ts1, td1 = pl.pallas_call(
        _tc_layer1,
        grid=(GRID,),
        in_specs=[
            pl.BlockSpec((RB, D), lambda i: (i, 0)),
            pl.BlockSpec((D, HC), lambda i: (0, 0)),
            pl.BlockSpec((HC, 16), lambda i: (0, 0)),
        ],
        out_specs=[
            pl.BlockSpec((RB, WS1), lambda i: (i, 0)),
            pl.BlockSpec((RB, WD), lambda i: (i, 0)),
        ],
        out_shape=[
            jax.ShapeDtypeStruct((N, WS1), jnp.float32),
            jax.ShapeDtypeStruct((N, WD), jnp.float32),
        ],
    )(x, W1, a1)

    acc1 = _make_edge_kernel(WS1, _edge1)(ts1, td1, srcl, dstl, z1)

    ts2, td2 = pl.pallas_call(
        _tc_layer2,
        grid=(GRID,),
        in_specs=[
            pl.BlockSpec((2, RB, WS1), lambda i: (0, i, 0)),
            pl.BlockSpec((RB, WS1), lambda i: (i, 0)),
            pl.BlockSpec((RB, WD), lambda i: (i, 0)),
            pl.BlockSpec((1, HC), lambda i: (0, 0)),
            pl.BlockSpec((H1, HC), lambda i: (0, 0)),
            pl.BlockSpec((HC, NCLS), lambda i: (0, 0)),
            pl.BlockSpec((NCLS, 2), lambda i: (0, 0)),
        ],
        out_specs=[
            pl.BlockSpec((RB, WS2), lambda i: (i, 0)),
            pl.BlockSpec((RB, WD), lambda i: (i, 0)),
        ],
        out_shape=[
            jax.ShapeDtypeStruct((N, WS2), jnp.float32),
            jax.ShapeDtypeStruct((N, WD), jnp.float32),
        ],
    )(acc1, ts1, td1, b1.reshape(1, HC), e8, W2, a2)

    acc2 = _make_edge_kernel(WS2, _edge2)(ts2, td2, srcl, dstl, z2)

    out = pl.pallas_call(
        _tc_final,
        grid=(GRID,),
        in_specs=[
            pl.BlockSpec((2, RB, WS2), lambda i: (0, i, 0)),
            pl.BlockSpec((RB, WS2), lambda i: (i, 0)),
            pl.BlockSpec((RB, WD), lambda i: (i, 0)),
            pl.BlockSpec((1, NCLS), lambda i: (0, 0)),
        ],
        out_specs=pl.BlockSpec((RB, NCLS), lambda i: (i, 0)),
        out_shape=jax.ShapeDtypeStruct((N, NCLS), jnp.float32),
    )(acc2, ts2, td2, b2.reshape(1, NCLS))

    return out


# speedup vs baseline: 2.4191x; 1.2669x over previous
import functools

import jax
import jax.numpy as jnp
from jax import lax
from jax.experimental import pallas as pl
from jax.experimental.pallas import tpu as pltpu
from jax.experimental.pallas import tpu_sc as plsc

N = 10000
D = 128
H1 = 8
C1 = 8
HC = H1 * C1
NCLS = 40
RB = 400
GRID = N // RB
WS1 = 80
WD = 16
WS2 = 48
NW = 32
B = 128
SPW = N // 16
ROWS = 2500
CHB = ROWS // NW
XTRA = ROWS - CHB * NW
CHMAX = CHB + XTRA // 2


def _lrelu(x):
    return jnp.where(x >= 0.0, x, 0.2 * x)


def _tc_layer1(x_ref, w_ref, a_ref, ts_ref, td_ref):
    h = jnp.dot(x_ref[...], w_ref[...], preferred_element_type=jnp.float32)
    aa = jnp.dot(h, a_ref[...], preferred_element_type=jnp.float32)
    z8 = jnp.zeros((RB, 8), jnp.float32)
    ts_ref[...] = jnp.concatenate([h, aa[:, :H1], z8], axis=1)
    td_ref[...] = jnp.concatenate([aa[:, H1:], z8], axis=1)


def _tc_layer2(acc_ref, ts1_ref, td1_ref, b1_ref, e8_ref, w2_ref, a2_ref,
               ts_ref, td_ref):
    a = acc_ref[0] + acc_ref[1]
    h = ts1_ref[:, :HC]
    exs = jnp.exp(_lrelu(ts1_ref[:, HC:HC + H1] + td1_ref[:, :H1]))
    den = a[:, HC:HC + H1] + exs
    num = a[:, :HC] + h * jnp.dot(exs, e8_ref[...],
                                  preferred_element_type=jnp.float32)
    r = 1.0 / (den + 1e-16)
    o = num * jnp.dot(r, e8_ref[...], preferred_element_type=jnp.float32)
    o = _lrelu(o + b1_ref[...])
    h2 = jnp.dot(o, w2_ref[...], preferred_element_type=jnp.float32)
    aa = jnp.dot(h2, a2_ref[...], preferred_element_type=jnp.float32)
    ts_ref[...] = jnp.concatenate(
        [aa[:, 0:1], jnp.zeros((RB, 7), jnp.float32), h2], axis=1)
    td_ref[...] = jnp.concatenate(
        [aa[:, 1:2], jnp.zeros((RB, 15), jnp.float32)], axis=1)


def _tc_final(acc_ref, ts2_ref, td2_ref, b2_ref, out_ref):
    a = acc_ref[0] + acc_ref[1]
    h2 = ts2_ref[:, 8:8 + NCLS]
    exs = jnp.exp(_lrelu(ts2_ref[:, 0:1] + td2_ref[:, 0:1]))
    den = a[:, 0:1] + exs
    num = a[:, 8:8 + NCLS] + h2 * exs
    o = num / (den + 1e-16) + b2_ref[...]
    m = jnp.max(o, axis=1, keepdims=True)
    s = o - m
    out_ref[...] = s - jnp.log(jnp.sum(jnp.exp(s), axis=1, keepdims=True))


def _make_edge_kernel(ws, per_edge):
    mesh = plsc.VectorSubcoreMesh(core_axis_name="c", subcore_axis_name="s")

    @functools.partial(
        pl.kernel,
        out_type=jax.ShapeDtypeStruct((2, N, ws), jnp.float32),
        mesh=mesh,
        scratch_types=[
            pltpu.VMEM_SHARED((N, ws), jnp.float32),
            pltpu.VMEM((CHMAX, B), jnp.int32),
            pltpu.VMEM((CHMAX, B), jnp.int32),
            (pltpu.VMEM((B, ws), jnp.float32),) * 2,
            (pltpu.VMEM((B, WD), jnp.float32),) * 2,
            (pltpu.VMEM((B, ws), jnp.float32),) * 2,
            (pltpu.SemaphoreType.DMA,) * 2,
            (pltpu.SemaphoreType.DMA,) * 2,
        ],
        compiler_params=pltpu.CompilerParams(
            use_tc_tiling_on_sc=False, needs_layout_passes=False),
    )
    def k(tsrc, tad, srcl, dstl, zrows, acc_out,
          acc_sh, sidxa, didxa, srows, drows, msg, semg, sems):
        c = lax.axis_index("c")
        s = lax.axis_index("s")
        wid = s * 2 + c
        start = CHB * wid + 2 * jnp.minimum(wid, XTRA // 2)
        nch = CHB + jnp.where(wid < XTRA // 2, XTRA // 2, 0)
        pltpu.sync_copy(zrows.at[pl.ds(s * SPW, SPW)],
                        acc_sh.at[pl.ds(s * SPW, SPW)])
        pltpu.sync_copy(srcl.at[pl.ds(start, CHMAX)], sidxa)
        pltpu.sync_copy(dstl.at[pl.ds(start, CHMAX)], didxa)
        plsc.subcore_barrier()

        def issue_gathers(t, p):
            pltpu.async_copy(tsrc.at[sidxa.at[t]], srows[p], semg[p])
            pltpu.async_copy(tad.at[didxa.at[t]], drows[p], semg[p])

        def wait_gathers(t, p):
            pltpu.make_async_copy(
                tsrc.at[sidxa.at[t]], srows[p], semg[p]).wait()
            pltpu.make_async_copy(
                tad.at[didxa.at[t]], drows[p], semg[p]).wait()

        def compute(p):
            def edge(e, carry):
                per_edge(e, srows[p], drows[p], msg[p])
                return carry

            lax.fori_loop(0, B, edge, 0)

        def issue_scatter(t, p):
            pltpu.async_copy(msg[p], acc_sh.at[didxa.at[t]], sems[p],
                             add=True)

        def wait_scatter(t, p):
            pltpu.make_async_copy(
                msg[p], acc_sh.at[didxa.at[t]], sems[p]).wait()

        issue_gathers(0, 0)
        issue_gathers(1, 1)
        wait_gathers(0, 0)
        compute(0)
        issue_scatter(0, 0)

        def pair(i, carry):
            t = 2 * i + 1
            issue_gathers(t + 1, 0)
            wait_gathers(t, 1)
            compute(1)
            wait_scatter(t - 1, 0)
            issue_scatter(t, 1)
            issue_gathers(t + 2, 1)
            wait_gathers(t + 1, 0)
            compute(0)
            wait_scatter(t, 1)
            issue_scatter(t + 1, 0)
            return carry

        lax.fori_loop(0, (nch - 2) // 2, pair, 0)
        wait_gathers(nch - 1, 1)
        compute(1)
        wait_scatter(nch - 2, 0)
        issue_scatter(nch - 1, 1)
        wait_scatter(nch - 1, 1)
        plsc.subcore_barrier()
        pltpu.sync_copy(acc_sh.at[pl.ds(s * SPW, SPW)],
                        acc_out.at[c, pl.ds(s * SPW, SPW)])

    return k


def _make_edge_kernel2():
    ws = WS2
    mesh = plsc.VectorSubcoreMesh(core_axis_name="c", subcore_axis_name="s")

    @functools.partial(
        pl.kernel,
        out_type=jax.ShapeDtypeStruct((2, N, ws), jnp.float32),
        mesh=mesh,
        scratch_types=[
            pltpu.VMEM_SHARED((N, ws), jnp.float32),
            pltpu.VMEM((CHMAX, B), jnp.int32),
            pltpu.VMEM((CHMAX, B), jnp.int32),
            pltpu.VMEM((N,), jnp.float32),
            (pltpu.VMEM((B, ws), jnp.float32),) * 2,
            (pltpu.VMEM((B, ws), jnp.float32),) * 2,
            (pltpu.SemaphoreType.DMA,) * 2,
            (pltpu.SemaphoreType.DMA,) * 2,
        ],
        compiler_params=pltpu.CompilerParams(
            use_tc_tiling_on_sc=False, needs_layout_passes=False),
    )
    def k(tsrc, tadv, srcl, dstl, zrows, acc_out,
          acc_sh, sidxa, didxa, advt, srows, msg, semg, sems):
        c = lax.axis_index("c")
        s = lax.axis_index("s")
        wid = s * 2 + c
        start = CHB * wid + 2 * jnp.minimum(wid, XTRA // 2)
        nch = CHB + jnp.where(wid < XTRA // 2, XTRA // 2, 0)
        pltpu.sync_copy(zrows.at[pl.ds(s * SPW, SPW)],
                        acc_sh.at[pl.ds(s * SPW, SPW)])
        pltpu.sync_copy(srcl.at[pl.ds(start, CHMAX)], sidxa)
        pltpu.sync_copy(dstl.at[pl.ds(start, CHMAX)], didxa)
        pltpu.sync_copy(tadv, advt)
        plsc.subcore_barrier()

        def issue_gathers(t, p):
            pltpu.async_copy(tsrc.at[sidxa.at[t]], srows[p], semg[p])

        def wait_gathers(t, p):
            pltpu.make_async_copy(
                tsrc.at[sidxa.at[t]], srows[p], semg[p]).wait()

        lane = lax.broadcasted_iota(jnp.int32, (16,), 0)
        one = jnp.full((16,), 1.0, jnp.float32)
        z16 = jnp.zeros((16,), jnp.int32)

        def compute(t, p):
            sr = srows[p]
            mg = msg[p]

            def group(g, carry):
                dvec = didxa[t, pl.ds(g * 16, 16)]
                adv = plsc.load_gather(advt, [dvec])
                for u in range(16):
                    e = g * 16 + u
                    s0 = sr[e, pl.ds(0, 16)]
                    al = s0 + _vgather(adv, jnp.full((16,), u, jnp.int32))
                    al = jnp.where(al >= 0.0, al, 0.2 * al)
                    bex = _vgather(jnp.exp(al), z16)
                    mg[e, pl.ds(0, 16)] = bex * jnp.where(lane == 0, one, s0)
                    mg[e, pl.ds(16, 16)] = bex * sr[e, pl.ds(16, 16)]
                    mg[e, pl.ds(32, 16)] = bex * sr[e, pl.ds(32, 16)]
                return carry

            lax.fori_loop(0, B // 16, group, 0)

        def issue_scatter(t, p):
            pltpu.async_copy(msg[p], acc_sh.at[didxa.at[t]], sems[p],
                             add=True)

        def wait_scatter(t, p):
            pltpu.make_async_copy(
                msg[p], acc_sh.at[didxa.at[t]], sems[p]).wait()

        issue_gathers(0, 0)
        issue_gathers(1, 1)
        wait_gathers(0, 0)
        compute(0, 0)
        issue_scatter(0, 0)

        def pair(i, carry):
            t = 2 * i + 1
            issue_gathers(t + 1, 0)
            wait_gathers(t, 1)
            compute(t, 1)
            wait_scatter(t - 1, 0)
            issue_scatter(t, 1)
            issue_gathers(t + 2, 1)
            wait_gathers(t + 1, 0)
            compute(t + 1, 0)
            wait_scatter(t, 1)
            issue_scatter(t + 1, 0)
            return carry

        lax.fori_loop(0, (nch - 2) // 2, pair, 0)
        wait_gathers(nch - 1, 1)
        compute(nch - 1, 1)
        wait_scatter(nch - 2, 0)
        issue_scatter(nch - 1, 1)
        wait_scatter(nch - 1, 1)
        plsc.subcore_barrier()
        pltpu.sync_copy(acc_sh.at[pl.ds(s * SPW, SPW)],
                        acc_out.at[c, pl.ds(s * SPW, SPW)])

    return k


_GDN = lax.GatherDimensionNumbers(
    offset_dims=(), collapsed_slice_dims=(0,), start_index_map=(0,))


def _vgather(x, idx):
    return lax.gather(x, idx[:, None], _GDN, (1,),
                      mode=lax.GatherScatterMode.PROMISE_IN_BOUNDS)


def _edge1(e, srows, drows, msg):
    ad = drows[e, :]
    av = srows[e, pl.ds(HC, 16)]
    al = av + ad
    al = jnp.where(al >= 0.0, al, 0.2 * al)
    ex = jnp.exp(al)
    hbase = lax.broadcasted_iota(jnp.int32, (16,), 0) // 8
    for j in range(4):
        sc = _vgather(ex, hbase + 2 * j)
        msg[e, pl.ds(j * 16, 16)] = srows[e, pl.ds(j * 16, 16)] * sc
    msg[e, pl.ds(HC, 16)] = ex


def _edge2(e, srows, drows, msg):
    ad = drows[e, :]
    s0 = srows[e, pl.ds(0, 16)]
    al = s0 + ad
    al = jnp.where(al >= 0.0, al, 0.2 * al)
    ex = jnp.exp(al)
    bex = _vgather(ex, jnp.zeros((16,), jnp.int32))
    lane = lax.broadcasted_iota(jnp.int32, (16,), 0)
    one = jnp.full((16,), 1.0, jnp.float32)
    msg[e, pl.ds(0, 16)] = bex * jnp.where(lane == 0, one, s0)
    msg[e, pl.ds(16, 16)] = bex * srows[e, pl.ds(16, 16)]
    msg[e, pl.ds(32, 16)] = bex * srows[e, pl.ds(32, 16)]


def kernel(x, edge_index, W1, att_src1, att_dst1, b1,
           W2, att_src2, att_dst2, b2):
    eye8 = jnp.eye(H1, dtype=jnp.float32)
    a_s = (eye8[:, None, :] * att_src1[:, :, None]).reshape(HC, H1)
    a_d = (eye8[:, None, :] * att_dst1[:, :, None]).reshape(HC, H1)
    a1 = jnp.concatenate([a_s, a_d], axis=1)
    e8 = jnp.kron(eye8, jnp.ones((1, C1), jnp.float32))
    a2 = jnp.concatenate([att_src2.T, att_dst2.T], axis=1)
    srcl = jnp.pad(edge_index[0].reshape(ROWS, B), ((0, 2), (0, 0)))
    dstl = jnp.pad(edge_index[1].reshape(ROWS, B), ((0, 2), (0, 0)))
    z1 = jnp.zeros((N, WS1), jnp.float32)
    z2 = jnp.zeros((N, WS2), jnp.float32)

    ts1, td1 = pl.pallas_call(
        _tc_layer1,
        grid=(GRID,),
        in_specs=[
            pl.BlockSpec((RB, D), lambda i: (i, 0)),
            pl.BlockSpec((D, HC), lambda i: (0, 0)),
            pl.BlockSpec((HC, 16), lambda i: (0, 0)),
        ],
        out_specs=[
            pl.BlockSpec((RB, WS1), lambda i: (i, 0)),
            pl.BlockSpec((RB, WD), lambda i: (i, 0)),
        ],
        out_shape=[
            jax.ShapeDtypeStruct((N, WS1), jnp.float32),
            jax.ShapeDtypeStruct((N, WD), jnp.float32),
        ],
    )(x, W1, a1)

    acc1 = _make_edge_kernel(WS1, _edge1)(ts1, td1, srcl, dstl, z1)

    ts2, td2 = pl.pallas_call(
        _tc_layer2,
        grid=(GRID,),
        in_specs=[
            pl.BlockSpec((2, RB, WS1), lambda i: (0, i, 0)),
            pl.BlockSpec((RB, WS1), lambda i: (i, 0)),
            pl.BlockSpec((RB, WD), lambda i: (i, 0)),
            pl.BlockSpec((1, HC), lambda i: (0, 0)),
            pl.BlockSpec((H1, HC), lambda i: (0, 0)),
            pl.BlockSpec((HC, NCLS), lambda i: (0, 0)),
            pl.BlockSpec((NCLS, 2), lambda i: (0, 0)),
        ],
        out_specs=[
            pl.BlockSpec((RB, WS2), lambda i: (i, 0)),
            pl.BlockSpec((RB, WD), lambda i: (i, 0)),
        ],
        out_shape=[
            jax.ShapeDtypeStruct((N, WS2), jnp.float32),
            jax.ShapeDtypeStruct((N, WD), jnp.float32),
        ],
    )(acc1, ts1, td1, b1.reshape(1, HC), e8, W2, a2)

    acc2 = _make_edge_kernel2()(ts2, td2[:, 0], srcl, dstl, z2)

    out = pl.pallas_call(
        _tc_final,
        grid=(GRID,),
        in_specs=[
            pl.BlockSpec((2, RB, WS2), lambda i: (0, i, 0)),
            pl.BlockSpec((RB, WS2), lambda i: (i, 0)),
            pl.BlockSpec((RB, WD), lambda i: (i, 0)),
            pl.BlockSpec((1, NCLS), lambda i: (0, 0)),
        ],
        out_specs=pl.BlockSpec((RB, NCLS), lambda i: (i, 0)),
        out_shape=jax.ShapeDtypeStruct((N, NCLS), jnp.float32),
    )(acc2, ts2, td2, b2.reshape(1, NCLS))

    return out
